# baseline (speedup 1.0000x reference)
import functools

import jax
import jax.numpy as jnp
from jax import lax
from jax.experimental import pallas as pl
from jax.experimental.pallas import tpu as pltpu
from jax.experimental.pallas import tpu_sc as plsc

N = 10000
E = 320000
DN = 128
DE = 16
DG = 128
H = 128

NC = 2
NS = 16
NW = NC * NS
EW = E // NW
K = 40
NCH = EW // K
NPAIR = NCH // 2
WT = 10
RPT = N // WT
VPR = DN // 16

_mesh = plsc.VectorSubcoreMesh(
    core_axis_name="c", subcore_axis_name="s", num_cores=NC, num_subcores=NS)


def _tc1_body(nf_ref, wsr_ref, wg_ref, g_ref, be1_ref, ps_ref, pr_ref):
    base = jnp.dot(g_ref[...], wg_ref[...],
                   preferred_element_type=jnp.float32) + be1_ref[...]
    p = jnp.dot(nf_ref[...], wsr_ref[...], preferred_element_type=jnp.float32)
    ps_ref[...] = (p[:, :H] + base).astype(jnp.bfloat16)
    pr_ref[...] = p[:, H:].astype(jnp.bfloat16)


def _tc1(nf, wsr, wg, g, be1):
    return pl.pallas_call(
        _tc1_body,
        out_shape=(jax.ShapeDtypeStruct((N, H), jnp.bfloat16),
                   jax.ShapeDtypeStruct((N, H), jnp.bfloat16)),
    )(nf, wsr, wg, g, be1)


def _sc1_body(ps_hbm, pr_hbm, snd_hbm, rcv_hbm, g_hbm,
              idxall_s, idxall_r, buf_s, buf_r, gbuf,
              isem, gsem, ssem):
    cid = lax.axis_index("c")
    sid = lax.axis_index("s")
    wid = sid * NC + cid
    base_e = wid * EW

    pltpu.async_copy(snd_hbm.at[pl.ds(base_e, EW)], idxall_s, isem)
    pltpu.async_copy(rcv_hbm.at[pl.ds(base_e, EW)], idxall_r, isem)
    pltpu.make_async_copy(snd_hbm.at[pl.ds(base_e, EW)], idxall_s,
                          isem).wait()
    pltpu.make_async_copy(rcv_hbm.at[pl.ds(base_e, EW)], idxall_r,
                          isem).wait()

    def start(c, slot):
        pltpu.async_copy(ps_hbm.at[idxall_s.at[pl.ds(c * K, K)]],
                         buf_s.at[slot], gsem.at[slot])
        pltpu.async_copy(pr_hbm.at[idxall_r.at[pl.ds(c * K, K)]],
                         buf_r.at[slot], gsem.at[slot])

    def wait_gathers(c, slot):
        pltpu.make_async_copy(ps_hbm.at[idxall_s.at[pl.ds(c * K, K)]],
                              buf_s.at[slot], gsem.at[slot]).wait()
        pltpu.make_async_copy(pr_hbm.at[idxall_r.at[pl.ds(c * K, K)]],
                              buf_r.at[slot], gsem.at[slot]).wait()

    def wait_store(c, slot):
        pltpu.make_async_copy(
            gbuf.at[slot], g_hbm.at[pl.ds(base_e + c * K, K)],
            ssem.at[slot]).wait()

    start(0, 0)
    start(1, 1)

    def pair(p, carry):
        for slot in range(2):
            c = 2 * p + slot
            wait_gathers(c, slot)

            @pl.when(p > 0)
            def _():
                wait_store(c - 2, slot)

            def vadd(i, carry2):
                for j in range(DN // 32):
                    sl = pl.ds(j * 32, 32)
                    gbuf[slot, i, sl] = buf_s[slot, i, sl] + buf_r[slot, i, sl]
                return carry2

            lax.fori_loop(0, K, vadd, 0)

            @pl.when(p < NPAIR - 1)
            def _():
                start(c + 2, slot)

            pltpu.async_copy(gbuf.at[slot],
                             g_hbm.at[pl.ds(base_e + c * K, K)],
                             ssem.at[slot])
        return carry

    lax.fori_loop(0, NPAIR, pair, 0)
    wait_store(NCH - 2, 0)
    wait_store(NCH - 1, 1)


def _sc1(ps, pr, snd, rcv):
    return pl.kernel(
        _sc1_body,
        out_type=jax.ShapeDtypeStruct((E, DN), jnp.bfloat16),
        mesh=_mesh,
        compiler_params=pltpu.CompilerParams(use_tc_tiling_on_sc=False),
        scratch_types=[
            pltpu.VMEM((EW,), jnp.int32),
            pltpu.VMEM((EW,), jnp.int32),
            pltpu.VMEM((2, K, DN), jnp.bfloat16),
            pltpu.VMEM((2, K, DN), jnp.bfloat16),
            pltpu.VMEM((2, K, DN), jnp.bfloat16),
            pltpu.SemaphoreType.DMA,
            pltpu.SemaphoreType.DMA((2,)),
            pltpu.SemaphoreType.DMA((2,)),
        ],
    )(ps, pr, snd, rcv)


EB = 4000


def _tc2_body(g_ref, ef_ref, w1e_ref, w2_ref, be2_ref, ne_ref):
    he = jnp.maximum(
        g_ref[...].astype(jnp.float32)
        + jnp.dot(ef_ref[...], w1e_ref[...],
                  preferred_element_type=jnp.float32), 0.0)
    ne_ref[...] = jnp.dot(he, w2_ref[...],
                          preferred_element_type=jnp.float32) + be2_ref[...]


def _tc2(g, ef, w1e, w2, be2):
    return pl.pallas_call(
        _tc2_body,
        grid=(E // EB,),
        in_specs=[
            pl.BlockSpec((EB, DN), lambda i: (i, 0)),
            pl.BlockSpec((EB, DE), lambda i: (i, 0)),
            pl.BlockSpec((DE, H), lambda i: (0, 0)),
            pl.BlockSpec((H, DE), lambda i: (0, 0)),
            pl.BlockSpec((1, DE), lambda i: (0, 0)),
        ],
        out_specs=pl.BlockSpec((EB, DE), lambda i: (i, 0)),
        out_shape=jax.ShapeDtypeStruct((E, DE), jnp.float32),
    )(g, ef, w1e, w2, be2)


def _sc2_body(ne_hbm, snd_hbm, rcv_hbm, aggp_hbm,
              idx_s, idx_r, nebuf, zbuf, acc_s, acc_r, lsem, xsem):
    cid = lax.axis_index("c")
    sid = lax.axis_index("s")
    wid = sid * NC + cid
    base_e = wid * EW

    @pl.when(sid < WT)
    def _zero():
        def zrow(i, carry):
            zbuf[i, :] = jnp.zeros((16,), jnp.float32)
            return carry

        lax.fori_loop(0, RPT, zrow, 0)
        pltpu.sync_copy(zbuf, acc_s.at[pl.ds(sid * RPT, RPT)])
        pltpu.sync_copy(zbuf, acc_r.at[pl.ds(sid * RPT, RPT)])

    plsc.subcore_barrier()

    def start_loads(c, slot):
        off = base_e + c * K
        pltpu.async_copy(snd_hbm.at[pl.ds(off, K)], idx_s.at[slot],
                         lsem.at[slot])
        pltpu.async_copy(rcv_hbm.at[pl.ds(off, K)], idx_r.at[slot],
                         lsem.at[slot])
        pltpu.async_copy(ne_hbm.at[pl.ds(off, K)], nebuf.at[slot],
                         lsem.at[slot])

    def wait_loads(c, slot):
        off = base_e + c * K
        pltpu.make_async_copy(snd_hbm.at[pl.ds(off, K)], idx_s.at[slot],
                              lsem.at[slot]).wait()
        pltpu.make_async_copy(rcv_hbm.at[pl.ds(off, K)], idx_r.at[slot],
                              lsem.at[slot]).wait()
        pltpu.make_async_copy(ne_hbm.at[pl.ds(off, K)], nebuf.at[slot],
                              lsem.at[slot]).wait()

    def issue_scatters(slot):
        pltpu.async_copy(nebuf.at[slot], acc_s.at[idx_s.at[slot]],
                         xsem.at[slot], add=True)
        pltpu.async_copy(nebuf.at[slot], acc_r.at[idx_r.at[slot]],
                         xsem.at[slot], add=True)

    def wait_scatters(slot):
        pltpu.make_async_copy(nebuf.at[slot], acc_s.at[idx_s.at[slot]],
                              xsem.at[slot]).wait()
        pltpu.make_async_copy(nebuf.at[slot], acc_r.at[idx_r.at[slot]],
                              xsem.at[slot]).wait()

    start_loads(0, 0)
    start_loads(1, 1)

    def quad(p, carry):
        for q in range(4):
            c = 4 * p + q
            slot_pf = (q + 2) % 4

            @pl.when((c >= 2) & (c + 2 < NCH))
            def _():
                wait_scatters(slot_pf)

            @pl.when(c + 2 < NCH)
            def _():
                start_loads(c + 2, slot_pf)

            @pl.when(c < NCH)
            def _():
                wait_loads(c, q)
                issue_scatters(q)
        return carry

    lax.fori_loop(0, (NCH + 3) // 4, quad, 0)
    for cc in range(NCH - 4, NCH):
        wait_scatters(cc % 4)
    plsc.subcore_barrier()

    @pl.when(sid < WT)
    def _writeout():
        sl = pl.ds(sid * RPT, RPT)
        pltpu.sync_copy(acc_s.at[sl], aggp_hbm.at[cid, 0, sl])
        pltpu.sync_copy(acc_r.at[sl], aggp_hbm.at[cid, 1, sl])


def _sc2(ne, snd, rcv):
    return pl.kernel(
        _sc2_body,
        out_type=jax.ShapeDtypeStruct((NC, 2, N, DE), jnp.float32),
        mesh=_mesh,
        compiler_params=pltpu.CompilerParams(use_tc_tiling_on_sc=False),
        scratch_types=[
            pltpu.VMEM((4, K), jnp.int32),
            pltpu.VMEM((4, K), jnp.int32),
            pltpu.VMEM((4, K, DE), jnp.float32),
            pltpu.VMEM((RPT, DE), jnp.float32),
            pltpu.VMEM_SHARED((N, DE), jnp.float32),
            pltpu.VMEM_SHARED((N, DE), jnp.float32),
            pltpu.SemaphoreType.DMA((4,)),
            pltpu.SemaphoreType.DMA((4,)),
        ],
    )(ne, snd, rcv)


def _tc3_body(nf_ref, aggp_ref, g_ref,
              wn1n_ref, wn1s_ref, wn1r_ref, wn1g_ref, bn1_ref,
              wn2_ref, bn2_ref,
              wg1g_ref, wg1n_ref, wg1e_ref, bg1_ref, wg2_ref, bg2_ref,
              nn_ref, ng_ref):
    agg_s = aggp_ref[0, 0] + aggp_ref[1, 0]
    agg_r = aggp_ref[0, 1] + aggp_ref[1, 1]
    dot = functools.partial(jnp.dot, preferred_element_type=jnp.float32)
    hn = jnp.maximum(
        dot(nf_ref[...], wn1n_ref[...]) + dot(agg_s, wn1s_ref[...])
        + dot(agg_r, wn1r_ref[...])
        + (dot(g_ref[...], wn1g_ref[...]) + bn1_ref[...]), 0.0)
    nn = dot(hn, wn2_ref[...]) + bn2_ref[...]
    nn_ref[...] = nn
    n2g = jnp.sum(nn, axis=0, keepdims=True)
    e2g = jnp.sum(agg_s, axis=0, keepdims=True)
    hg = jnp.maximum(
        dot(g_ref[...], wg1g_ref[...]) + dot(n2g, wg1n_ref[...])
        + dot(e2g, wg1e_ref[...]) + bg1_ref[...], 0.0)
    ng_ref[...] = dot(hg, wg2_ref[...]) + bg2_ref[...]


def _tc3(nf, aggp, g, wn1n, wn1s, wn1r, wn1g, bn1, wn2, bn2,
         wg1g, wg1n, wg1e, bg1, wg2, bg2):
    return pl.pallas_call(
        _tc3_body,
        out_shape=(jax.ShapeDtypeStruct((N, DN), jnp.float32),
                   jax.ShapeDtypeStruct((1, DG), jnp.float32)),
    )(nf, aggp, g, wn1n, wn1s, wn1r, wn1g, bn1, wn2, bn2,
      wg1g, wg1n, wg1e, bg1, wg2, bg2)


def kernel(node_features, edge_features, global_features, senders, receivers,
           We1, be1, We2, be2, Wn1, bn1, Wn2, bn2, Wg1, bg1, Wg2, bg2):
    w1e = We1[:DE]
    wsr = jnp.concatenate(
        [We1[DE:DE + DN], We1[DE + DN:DE + 2 * DN]], axis=1)
    wg = We1[DE + 2 * DN:]
    ps, pr = _tc1(node_features, wsr, wg, global_features,
                  be1.reshape(1, H))
    g_gath = _sc1(ps, pr, senders, receivers)
    ne = _tc2(g_gath, edge_features, w1e, We2, be2.reshape(1, DE))
    aggp = _sc2(ne, senders, receivers)
    wn1n = Wn1[:DN]
    wn1s = Wn1[DN:DN + DE]
    wn1r = Wn1[DN + DE:DN + 2 * DE]
    wn1g = Wn1[DN + 2 * DE:]
    wg1g = Wg1[:DG]
    wg1n = Wg1[DG:DG + DN]
    wg1e = Wg1[DG + DN:]
    nn, ng = _tc3(node_features, aggp, global_features,
                  wn1n, wn1s, wn1r, wn1g, bn1.reshape(1, H), Wn2,
                  bn2.reshape(1, DN),
                  wg1g, wg1n, wg1e, bg1.reshape(1, H), Wg2,
                  bg2.reshape(1, DG))
    return (nn, ne, ng)


# speedup vs baseline: 1.2977x; 1.2977x over previous
import functools

import jax
import jax.numpy as jnp
from jax import lax
from jax.experimental import pallas as pl
from jax.experimental.pallas import tpu as pltpu
from jax.experimental.pallas import tpu_sc as plsc

N = 10000
E = 320000
DN = 128
DE = 16
DG = 128
H = 128

NC = 2
NS = 16
NW = NC * NS
EW = E // NW
K = 40
NCH = EW // K
NPAIR = NCH // 2
WT = 10
RPT = N // WT
TW = DN // 2
VPR = TW // 16

_mesh = plsc.VectorSubcoreMesh(
    core_axis_name="c", subcore_axis_name="s", num_cores=NC, num_subcores=NS)


def _tc1_body(nf_ref, wsr_ref, wg_ref, g_ref, be1_ref, ps_ref, pr_ref):
    base = jnp.dot(g_ref[...], wg_ref[...],
                   preferred_element_type=jnp.float32) + be1_ref[...]
    p = jnp.dot(nf_ref[...], wsr_ref[...], preferred_element_type=jnp.float32)

    def pack(x):
        u = lax.bitcast_convert_type(
            x.astype(jnp.bfloat16), jnp.uint16).astype(jnp.uint32)
        word = u[:, :TW] | (u[:, TW:] << 16)
        return lax.bitcast_convert_type(word, jnp.float32)

    ps_ref[...] = pack(p[:, :H] + base)
    pr_ref[...] = pack(p[:, H:])


def _tc1(nf, wsr, wg, g, be1):
    return pl.pallas_call(
        _tc1_body,
        out_shape=(jax.ShapeDtypeStruct((N, TW), jnp.float32),
                   jax.ShapeDtypeStruct((N, TW), jnp.float32)),
    )(nf, wsr, wg, g, be1)


def _sc1_body(ps_hbm, pr_hbm, snd_hbm, rcv_hbm, g_hbm,
              idxall_s, idxall_r, buf_s, buf_r, gbuf,
              isem, gsem, ssem):
    cid = lax.axis_index("c")
    sid = lax.axis_index("s")
    wid = sid * NC + cid
    base_e = wid * EW

    pltpu.async_copy(snd_hbm.at[pl.ds(base_e, EW)], idxall_s, isem)
    pltpu.async_copy(rcv_hbm.at[pl.ds(base_e, EW)], idxall_r, isem)
    pltpu.make_async_copy(snd_hbm.at[pl.ds(base_e, EW)], idxall_s,
                          isem).wait()
    pltpu.make_async_copy(rcv_hbm.at[pl.ds(base_e, EW)], idxall_r,
                          isem).wait()

    def start(c, slot):
        pltpu.async_copy(ps_hbm.at[idxall_s.at[pl.ds(c * K, K)]],
                         buf_s.at[slot], gsem.at[slot])
        pltpu.async_copy(pr_hbm.at[idxall_r.at[pl.ds(c * K, K)]],
                         buf_r.at[slot], gsem.at[slot])

    def wait_gathers(c, slot):
        pltpu.make_async_copy(ps_hbm.at[idxall_s.at[pl.ds(c * K, K)]],
                              buf_s.at[slot], gsem.at[slot]).wait()
        pltpu.make_async_copy(pr_hbm.at[idxall_r.at[pl.ds(c * K, K)]],
                              buf_r.at[slot], gsem.at[slot]).wait()

    def wait_store(c, slot):
        pltpu.make_async_copy(
            gbuf.at[slot], g_hbm.at[pl.ds(base_e + c * K, K)],
            ssem.at[slot]).wait()

    start(0, 0)
    start(1, 1)

    def pair(p, carry):
        for slot in range(2):
            c = 2 * p + slot
            wait_gathers(c, slot)

            @pl.when(p > 0)
            def _():
                wait_store(c - 2, slot)

            def vadd(i, carry2):
                for j in range(VPR):
                    sl = pl.ds(j * 16, 16)
                    vs = plsc.bitcast(buf_s[slot, i, sl], jnp.bfloat16)
                    vr = plsc.bitcast(buf_r[slot, i, sl], jnp.bfloat16)
                    gbuf[slot, i, sl] = plsc.bitcast(vs + vr, jnp.float32)
                return carry2

            lax.fori_loop(0, K, vadd, 0)

            @pl.when(p < NPAIR - 1)
            def _():
                start(c + 2, slot)

            pltpu.async_copy(gbuf.at[slot],
                             g_hbm.at[pl.ds(base_e + c * K, K)],
                             ssem.at[slot])
        return carry

    lax.fori_loop(0, NPAIR, pair, 0)
    wait_store(NCH - 2, 0)
    wait_store(NCH - 1, 1)


def _sc1(ps, pr, snd, rcv):
    return pl.kernel(
        _sc1_body,
        out_type=jax.ShapeDtypeStruct((E, TW), jnp.float32),
        mesh=_mesh,
        compiler_params=pltpu.CompilerParams(use_tc_tiling_on_sc=False,
                                             needs_layout_passes=False),
        scratch_types=[
            pltpu.VMEM((EW,), jnp.int32),
            pltpu.VMEM((EW,), jnp.int32),
            pltpu.VMEM((2, K, TW), jnp.float32),
            pltpu.VMEM((2, K, TW), jnp.float32),
            pltpu.VMEM((2, K, TW), jnp.float32),
            pltpu.SemaphoreType.DMA,
            pltpu.SemaphoreType.DMA((2,)),
            pltpu.SemaphoreType.DMA((2,)),
        ],
    )(ps, pr, snd, rcv)


EB = 4000


def _tc2_body(g_ref, ef_ref, w1e_ref, w2_ref, be2_ref, ne_ref):
    w = lax.bitcast_convert_type(g_ref[...], jnp.uint32)
    lo = lax.bitcast_convert_type(w << 16, jnp.float32)
    hi = lax.bitcast_convert_type(w & jnp.uint32(0xFFFF0000), jnp.float32)
    gfull = jnp.concatenate([lo, hi], axis=-1)
    he = jnp.maximum(
        gfull + jnp.dot(ef_ref[...], w1e_ref[...],
                        preferred_element_type=jnp.float32), 0.0)
    ne_ref[...] = jnp.dot(he, w2_ref[...],
                          preferred_element_type=jnp.float32) + be2_ref[...]


def _tc2(g, ef, w1e, w2, be2):
    return pl.pallas_call(
        _tc2_body,
        grid=(E // EB,),
        in_specs=[
            pl.BlockSpec((EB, TW), lambda i: (i, 0)),
            pl.BlockSpec((EB, DE), lambda i: (i, 0)),
            pl.BlockSpec((DE, H), lambda i: (0, 0)),
            pl.BlockSpec((H, DE), lambda i: (0, 0)),
            pl.BlockSpec((1, DE), lambda i: (0, 0)),
        ],
        out_specs=pl.BlockSpec((EB, DE), lambda i: (i, 0)),
        out_shape=jax.ShapeDtypeStruct((E, DE), jnp.float32),
    )(g, ef, w1e, w2, be2)


def _sc2_body(ne_hbm, snd_hbm, rcv_hbm, aggp_hbm,
              idx_s, idx_r, nebuf, zbuf, acc_s, acc_r, lsem, xsem):
    cid = lax.axis_index("c")
    sid = lax.axis_index("s")
    wid = sid * NC + cid
    base_e = wid * EW

    @pl.when(sid < WT)
    def _zero():
        def zrow(i, carry):
            zbuf[i, :] = jnp.zeros((16,), jnp.float32)
            return carry

        lax.fori_loop(0, RPT, zrow, 0)
        pltpu.sync_copy(zbuf, acc_s.at[pl.ds(sid * RPT, RPT)])
        pltpu.sync_copy(zbuf, acc_r.at[pl.ds(sid * RPT, RPT)])

    plsc.subcore_barrier()

    def start_loads(c, slot):
        off = base_e + c * K
        pltpu.async_copy(snd_hbm.at[pl.ds(off, K)], idx_s.at[slot],
                         lsem.at[slot])
        pltpu.async_copy(rcv_hbm.at[pl.ds(off, K)], idx_r.at[slot],
                         lsem.at[slot])
        pltpu.async_copy(ne_hbm.at[pl.ds(off, K)], nebuf.at[slot],
                         lsem.at[slot])

    def wait_loads(c, slot):
        off = base_e + c * K
        pltpu.make_async_copy(snd_hbm.at[pl.ds(off, K)], idx_s.at[slot],
                              lsem.at[slot]).wait()
        pltpu.make_async_copy(rcv_hbm.at[pl.ds(off, K)], idx_r.at[slot],
                              lsem.at[slot]).wait()
        pltpu.make_async_copy(ne_hbm.at[pl.ds(off, K)], nebuf.at[slot],
                              lsem.at[slot]).wait()

    def issue_scatters(slot):
        pltpu.async_copy(nebuf.at[slot], acc_s.at[idx_s.at[slot]],
                         xsem.at[slot], add=True)
        pltpu.async_copy(nebuf.at[slot], acc_r.at[idx_r.at[slot]],
                         xsem.at[slot], add=True)

    def wait_scatters(slot):
        pltpu.make_async_copy(nebuf.at[slot], acc_s.at[idx_s.at[slot]],
                              xsem.at[slot]).wait()
        pltpu.make_async_copy(nebuf.at[slot], acc_r.at[idx_r.at[slot]],
                              xsem.at[slot]).wait()

    start_loads(0, 0)
    start_loads(1, 1)

    def quad(p, carry):
        for q in range(4):
            c = 4 * p + q
            slot_pf = (q + 2) % 4

            @pl.when((c >= 2) & (c + 2 < NCH))
            def _():
                wait_scatters(slot_pf)

            @pl.when(c + 2 < NCH)
            def _():
                start_loads(c + 2, slot_pf)

            @pl.when(c < NCH)
            def _():
                wait_loads(c, q)
                issue_scatters(q)
        return carry

    lax.fori_loop(0, (NCH + 3) // 4, quad, 0)
    for cc in range(NCH - 4, NCH):
        wait_scatters(cc % 4)
    plsc.subcore_barrier()

    @pl.when(sid < WT)
    def _writeout():
        sl = pl.ds(sid * RPT, RPT)
        pltpu.sync_copy(acc_s.at[sl], aggp_hbm.at[cid, 0, sl])
        pltpu.sync_copy(acc_r.at[sl], aggp_hbm.at[cid, 1, sl])


def _sc2(ne, snd, rcv):
    return pl.kernel(
        _sc2_body,
        out_type=jax.ShapeDtypeStruct((NC, 2, N, DE), jnp.float32),
        mesh=_mesh,
        compiler_params=pltpu.CompilerParams(use_tc_tiling_on_sc=False),
        scratch_types=[
            pltpu.VMEM((4, K), jnp.int32),
            pltpu.VMEM((4, K), jnp.int32),
            pltpu.VMEM((4, K, DE), jnp.float32),
            pltpu.VMEM((RPT, DE), jnp.float32),
            pltpu.VMEM_SHARED((N, DE), jnp.float32),
            pltpu.VMEM_SHARED((N, DE), jnp.float32),
            pltpu.SemaphoreType.DMA((4,)),
            pltpu.SemaphoreType.DMA((4,)),
        ],
    )(ne, snd, rcv)


def _tc3_body(nf_ref, aggp_ref, g_ref,
              wn1n_ref, wn1s_ref, wn1r_ref, wn1g_ref, bn1_ref,
              wn2_ref, bn2_ref,
              wg1g_ref, wg1n_ref, wg1e_ref, bg1_ref, wg2_ref, bg2_ref,
              nn_ref, ng_ref):
    agg_s = aggp_ref[0, 0] + aggp_ref[1, 0]
    agg_r = aggp_ref[0, 1] + aggp_ref[1, 1]
    dot = functools.partial(jnp.dot, preferred_element_type=jnp.float32)
    hn = jnp.maximum(
        dot(nf_ref[...], wn1n_ref[...]) + dot(agg_s, wn1s_ref[...])
        + dot(agg_r, wn1r_ref[...])
        + (dot(g_ref[...], wn1g_ref[...]) + bn1_ref[...]), 0.0)
    nn = dot(hn, wn2_ref[...]) + bn2_ref[...]
    nn_ref[...] = nn
    n2g = jnp.sum(nn, axis=0, keepdims=True)
    e2g = jnp.sum(agg_s, axis=0, keepdims=True)
    hg = jnp.maximum(
        dot(g_ref[...], wg1g_ref[...]) + dot(n2g, wg1n_ref[...])
        + dot(e2g, wg1e_ref[...]) + bg1_ref[...], 0.0)
    ng_ref[...] = dot(hg, wg2_ref[...]) + bg2_ref[...]


def _tc3(nf, aggp, g, wn1n, wn1s, wn1r, wn1g, bn1, wn2, bn2,
         wg1g, wg1n, wg1e, bg1, wg2, bg2):
    return pl.pallas_call(
        _tc3_body,
        out_shape=(jax.ShapeDtypeStruct((N, DN), jnp.float32),
                   jax.ShapeDtypeStruct((1, DG), jnp.float32)),
    )(nf, aggp, g, wn1n, wn1s, wn1r, wn1g, bn1, wn2, bn2,
      wg1g, wg1n, wg1e, bg1, wg2, bg2)


def kernel(node_features, edge_features, global_features, senders, receivers,
           We1, be1, We2, be2, Wn1, bn1, Wn2, bn2, Wg1, bg1, Wg2, bg2):
    w1e = We1[:DE]
    wsr = jnp.concatenate(
        [We1[DE:DE + DN], We1[DE + DN:DE + 2 * DN]], axis=1)
    wg = We1[DE + 2 * DN:]
    ps, pr = _tc1(node_features, wsr, wg, global_features,
                  be1.reshape(1, H))
    g_gath = _sc1(ps, pr, senders, receivers)
    ne = _tc2(g_gath, edge_features, w1e, We2, be2.reshape(1, DE))
    aggp = _sc2(ne, senders, receivers)
    wn1n = Wn1[:DN]
    wn1s = Wn1[DN:DN + DE]
    wn1r = Wn1[DN + DE:DN + 2 * DE]
    wn1g = Wn1[DN + 2 * DE:]
    wg1g = Wg1[:DG]
    wg1n = Wg1[DG:DG + DN]
    wg1e = Wg1[DG + DN:]
    nn, ng = _tc3(node_features, aggp, global_features,
                  wn1n, wn1s, wn1r, wn1g, bn1.reshape(1, H), Wn2,
                  bn2.reshape(1, DN),
                  wg1g, wg1n, wg1e, bg1.reshape(1, H), Wg2,
                  bg2.reshape(1, DG))
    return (nn, ne, ng)


# speedup vs baseline: 1.4503x; 1.1176x over previous
import functools

import jax
import jax.numpy as jnp
from jax import lax
from jax.experimental import pallas as pl
from jax.experimental.pallas import tpu as pltpu
from jax.experimental.pallas import tpu_sc as plsc

N = 10000
E = 320000
DN = 128
DE = 16
DG = 128
H = 128

NC = 2
NS = 16
NW = NC * NS
EW = E // NW
K = 40
NCH = EW // K
NQUAD = (NCH + 3) // 4
WT = 10
RPT = N // WT
VPR = DN // 16

_mesh = plsc.VectorSubcoreMesh(
    core_axis_name="c", subcore_axis_name="s", num_cores=NC, num_subcores=NS)


def _tc1_body(nf_ref, wsr_ref, wg_ref, g_ref, be1_ref, ps_ref, pr_ref):
    base = jnp.dot(g_ref[...], wg_ref[...],
                   preferred_element_type=jnp.float32) + be1_ref[...]
    p = jnp.dot(nf_ref[...], wsr_ref[...], preferred_element_type=jnp.float32)
    ps_ref[...] = p[:, :H] + base
    pr_ref[...] = p[:, H:]


def _tc1(nf, wsr, wg, g, be1):
    return pl.pallas_call(
        _tc1_body,
        out_shape=(jax.ShapeDtypeStruct((N, H), jnp.float32),
                   jax.ShapeDtypeStruct((N, H), jnp.float32)),
    )(nf, wsr, wg, g, be1)


def _sc1_body(ps_hbm, pr_hbm, snd_hbm, rcv_hbm, g_hbm,
              idxall_s, idxall_r, buf_s, buf_r,
              isem, gsem, ssem):
    cid = lax.axis_index("c")
    sid = lax.axis_index("s")
    wid = sid * NC + cid
    base_e = wid * EW

    pltpu.async_copy(snd_hbm.at[pl.ds(base_e, EW)], idxall_s, isem)
    pltpu.async_copy(rcv_hbm.at[pl.ds(base_e, EW)], idxall_r, isem)
    pltpu.make_async_copy(snd_hbm.at[pl.ds(base_e, EW)], idxall_s,
                          isem).wait()
    pltpu.make_async_copy(rcv_hbm.at[pl.ds(base_e, EW)], idxall_r,
                          isem).wait()

    def start_gathers(c, slot):
        pltpu.async_copy(ps_hbm.at[idxall_s.at[pl.ds(c * K, K)]],
                         buf_s.at[slot], gsem.at[slot])
        pltpu.async_copy(pr_hbm.at[idxall_r.at[pl.ds(c * K, K)]],
                         buf_r.at[slot], gsem.at[slot])

    def wait_gathers(c, slot):
        pltpu.make_async_copy(ps_hbm.at[idxall_s.at[pl.ds(c * K, K)]],
                              buf_s.at[slot], gsem.at[slot]).wait()
        pltpu.make_async_copy(pr_hbm.at[idxall_r.at[pl.ds(c * K, K)]],
                              buf_r.at[slot], gsem.at[slot]).wait()

    def wait_store(c, slot):
        pltpu.make_async_copy(
            buf_s.at[slot], g_hbm.at[pl.ds(base_e + c * K, K)],
            ssem.at[slot]).wait()

    start_gathers(0, 0)
    start_gathers(1, 1)

    def quad(p, carry):
        for q in range(4):
            c = 4 * p + q
            slot_pf = (q + 2) % 4

            @pl.when((c >= 2) & (c + 2 < NCH))
            def _():
                wait_store(c - 2, slot_pf)

            @pl.when(c + 2 < NCH)
            def _():
                start_gathers(c + 2, slot_pf)

            @pl.when(c < NCH)
            def _():
                wait_gathers(c, q)

                def vadd(i, carry2):
                    for j in range(VPR):
                        sl = pl.ds(j * 16, 16)
                        buf_s[q, i, sl] = buf_s[q, i, sl] + buf_r[q, i, sl]
                    return carry2

                lax.fori_loop(0, K, vadd, 0)
                pltpu.async_copy(buf_s.at[q],
                                 g_hbm.at[pl.ds(base_e + c * K, K)],
                                 ssem.at[q])
        return carry

    lax.fori_loop(0, NQUAD, quad, 0)
    for cc in range(NCH - 4, NCH):
        wait_store(cc, cc % 4)


def _sc1(ps, pr, snd, rcv):
    return pl.kernel(
        _sc1_body,
        out_type=jax.ShapeDtypeStruct((E, DN), jnp.float32),
        mesh=_mesh,
        compiler_params=pltpu.CompilerParams(use_tc_tiling_on_sc=False),
        scratch_types=[
            pltpu.VMEM((EW,), jnp.int32),
            pltpu.VMEM((EW,), jnp.int32),
            pltpu.VMEM((4, K, DN), jnp.float32),
            pltpu.VMEM((4, K, DN), jnp.float32),
            pltpu.SemaphoreType.DMA,
            pltpu.SemaphoreType.DMA((4,)),
            pltpu.SemaphoreType.DMA((4,)),
        ],
    )(ps, pr, snd, rcv)


EB = 4000


def _tc2_body(g_ref, ef_ref, w1e_ref, w2_ref, be2_ref, ne_ref):
    he = jnp.maximum(
        g_ref[...] + jnp.dot(ef_ref[...], w1e_ref[...],
                             preferred_element_type=jnp.float32), 0.0)
    ne_ref[...] = jnp.dot(he, w2_ref[...],
                          preferred_element_type=jnp.float32) + be2_ref[...]


def _tc2(g, ef, w1e, w2, be2):
    return pl.pallas_call(
        _tc2_body,
        grid=(E // EB,),
        in_specs=[
            pl.BlockSpec((EB, DN), lambda i: (i, 0)),
            pl.BlockSpec((EB, DE), lambda i: (i, 0)),
            pl.BlockSpec((DE, H), lambda i: (0, 0)),
            pl.BlockSpec((H, DE), lambda i: (0, 0)),
            pl.BlockSpec((1, DE), lambda i: (0, 0)),
        ],
        out_specs=pl.BlockSpec((EB, DE), lambda i: (i, 0)),
        out_shape=jax.ShapeDtypeStruct((E, DE), jnp.float32),
    )(g, ef, w1e, w2, be2)


def _sc2_body(ne_hbm, snd_hbm, rcv_hbm, aggp_hbm,
              idx_s, idx_r, nebuf, zbuf, acc_s, acc_r, lsem, xsem):
    cid = lax.axis_index("c")
    sid = lax.axis_index("s")
    wid = sid * NC + cid
    base_e = wid * EW

    @pl.when(sid < WT)
    def _zero():
        def zrow(i, carry):
            zbuf[i, :] = jnp.zeros((16,), jnp.float32)
            return carry

        lax.fori_loop(0, RPT, zrow, 0)
        pltpu.sync_copy(zbuf, acc_s.at[pl.ds(sid * RPT, RPT)])
        pltpu.sync_copy(zbuf, acc_r.at[pl.ds(sid * RPT, RPT)])

    plsc.subcore_barrier()

    def start_loads(c, slot):
        off = base_e + c * K
        pltpu.async_copy(snd_hbm.at[pl.ds(off, K)], idx_s.at[slot],
                         lsem.at[slot])
        pltpu.async_copy(rcv_hbm.at[pl.ds(off, K)], idx_r.at[slot],
                         lsem.at[slot])
        pltpu.async_copy(ne_hbm.at[pl.ds(off, K)], nebuf.at[slot],
                         lsem.at[slot])

    def wait_loads(c, slot):
        off = base_e + c * K
        pltpu.make_async_copy(snd_hbm.at[pl.ds(off, K)], idx_s.at[slot],
                              lsem.at[slot]).wait()
        pltpu.make_async_copy(rcv_hbm.at[pl.ds(off, K)], idx_r.at[slot],
                              lsem.at[slot]).wait()
        pltpu.make_async_copy(ne_hbm.at[pl.ds(off, K)], nebuf.at[slot],
                              lsem.at[slot]).wait()

    def issue_scatters(slot):
        pltpu.async_copy(nebuf.at[slot], acc_s.at[idx_s.at[slot]],
                         xsem.at[slot], add=True)
        pltpu.async_copy(nebuf.at[slot], acc_r.at[idx_r.at[slot]],
                         xsem.at[slot], add=True)

    def wait_scatters(slot):
        pltpu.make_async_copy(nebuf.at[slot], acc_s.at[idx_s.at[slot]],
                              xsem.at[slot]).wait()
        pltpu.make_async_copy(nebuf.at[slot], acc_r.at[idx_r.at[slot]],
                              xsem.at[slot]).wait()

    start_loads(0, 0)
    start_loads(1, 1)

    def quad(p, carry):
        for q in range(4):
            c = 4 * p + q
            slot_pf = (q + 2) % 4

            @pl.when((c >= 2) & (c + 2 < NCH))
            def _():
                wait_scatters(slot_pf)

            @pl.when(c + 2 < NCH)
            def _():
                start_loads(c + 2, slot_pf)

            @pl.when(c < NCH)
            def _():
                wait_loads(c, q)
                issue_scatters(q)
        return carry

    lax.fori_loop(0, NQUAD, quad, 0)
    for cc in range(NCH - 4, NCH):
        wait_scatters(cc % 4)
    plsc.subcore_barrier()

    @pl.when(sid < WT)
    def _writeout():
        sl = pl.ds(sid * RPT, RPT)
        pltpu.sync_copy(acc_s.at[sl], aggp_hbm.at[cid, 0, sl])
        pltpu.sync_copy(acc_r.at[sl], aggp_hbm.at[cid, 1, sl])


def _sc2(ne, snd, rcv):
    return pl.kernel(
        _sc2_body,
        out_type=jax.ShapeDtypeStruct((NC, 2, N, DE), jnp.float32),
        mesh=_mesh,
        compiler_params=pltpu.CompilerParams(use_tc_tiling_on_sc=False),
        scratch_types=[
            pltpu.VMEM((4, K), jnp.int32),
            pltpu.VMEM((4, K), jnp.int32),
            pltpu.VMEM((4, K, DE), jnp.float32),
            pltpu.VMEM((RPT, DE), jnp.float32),
            pltpu.VMEM_SHARED((N, DE), jnp.float32),
            pltpu.VMEM_SHARED((N, DE), jnp.float32),
            pltpu.SemaphoreType.DMA((4,)),
            pltpu.SemaphoreType.DMA((4,)),
        ],
    )(ne, snd, rcv)


def _tc3_body(nf_ref, aggp_ref, g_ref,
              wn1n_ref, wn1s_ref, wn1r_ref, wn1g_ref, bn1_ref,
              wn2_ref, bn2_ref,
              wg1g_ref, wg1n_ref, wg1e_ref, bg1_ref, wg2_ref, bg2_ref,
              nn_ref, ng_ref):
    agg_s = aggp_ref[0, 0] + aggp_ref[1, 0]
    agg_r = aggp_ref[0, 1] + aggp_ref[1, 1]
    dot = functools.partial(jnp.dot, preferred_element_type=jnp.float32)
    hn = jnp.maximum(
        dot(nf_ref[...], wn1n_ref[...]) + dot(agg_s, wn1s_ref[...])
        + dot(agg_r, wn1r_ref[...])
        + (dot(g_ref[...], wn1g_ref[...]) + bn1_ref[...]), 0.0)
    nn = dot(hn, wn2_ref[...]) + bn2_ref[...]
    nn_ref[...] = nn
    n2g = jnp.sum(nn, axis=0, keepdims=True)
    e2g = jnp.sum(agg_s, axis=0, keepdims=True)
    hg = jnp.maximum(
        dot(g_ref[...], wg1g_ref[...]) + dot(n2g, wg1n_ref[...])
        + dot(e2g, wg1e_ref[...]) + bg1_ref[...], 0.0)
    ng_ref[...] = dot(hg, wg2_ref[...]) + bg2_ref[...]


def _tc3(nf, aggp, g, wn1n, wn1s, wn1r, wn1g, bn1, wn2, bn2,
         wg1g, wg1n, wg1e, bg1, wg2, bg2):
    return pl.pallas_call(
        _tc3_body,
        out_shape=(jax.ShapeDtypeStruct((N, DN), jnp.float32),
                   jax.ShapeDtypeStruct((1, DG), jnp.float32)),
    )(nf, aggp, g, wn1n, wn1s, wn1r, wn1g, bn1, wn2, bn2,
      wg1g, wg1n, wg1e, bg1, wg2, bg2)


def kernel(node_features, edge_features, global_features, senders, receivers,
           We1, be1, We2, be2, Wn1, bn1, Wn2, bn2, Wg1, bg1, Wg2, bg2):
    w1e = We1[:DE]
    wsr = jnp.concatenate(
        [We1[DE:DE + DN], We1[DE + DN:DE + 2 * DN]], axis=1)
    wg = We1[DE + 2 * DN:]
    ps, pr = _tc1(node_features, wsr, wg, global_features,
                  be1.reshape(1, H))
    g_gath = _sc1(ps, pr, senders, receivers)
    ne = _tc2(g_gath, edge_features, w1e, We2, be2.reshape(1, DE))
    aggp = _sc2(ne, senders, receivers)
    wn1n = Wn1[:DN]
    wn1s = Wn1[DN:DN + DE]
    wn1r = Wn1[DN + DE:DN + 2 * DE]
    wn1g = Wn1[DN + 2 * DE:]
    wg1g = Wg1[:DG]
    wg1n = Wg1[DG:DG + DN]
    wg1e = Wg1[DG + DN:]
    nn, ng = _tc3(node_features, aggp, global_features,
                  wn1n, wn1s, wn1r, wn1g, bn1.reshape(1, H), Wn2,
                  bn2.reshape(1, DN),
                  wg1g, wg1n, wg1e, bg1.reshape(1, H), Wg2,
                  bg2.reshape(1, DG))
    return (nn, ne, ng)
